# baseline (speedup 1.0000x reference)
import functools

import jax
import jax.numpy as jnp
from jax import lax
from jax.experimental import pallas as pl
from jax.experimental.pallas import tpu as pltpu
from jax.experimental.pallas import tpu_sc as plsc

D_MODEL = 64
SCALE = 8.0
LANES = 16
LB = 128
PAIR = 2 * D_MODEL
FT = 8
TILE_E = FT * 128


@functools.cache
def _make_gather(seq: int, batch: int, vocab: int):
    info = plsc.get_sparse_core_info()
    NC, NS = info.num_cores, info.num_subcores
    NW = NC * NS
    n_bt = batch // LB
    assert n_bt == NW and seq % 2 == 0 and vocab % 2 == 0, (seq, batch, vocab)

    mesh = plsc.VectorSubcoreMesh(core_axis_name="c", subcore_axis_name="s")

    @functools.partial(
        pl.kernel,
        mesh=mesh,
        compiler_params=pltpu.CompilerParams(
            use_tc_tiling_on_sc=False, needs_layout_passes=False
        ),
        out_type=jax.ShapeDtypeStruct((seq, FT, n_bt, TILE_E), jnp.float32),
        scratch_types=[
            pltpu.VMEM((seq // 8, 8, LB), jnp.int32),
            pltpu.VMEM((seq // 8, 8, LB), jnp.int32),
            pltpu.VMEM((LB, PAIR), jnp.float32),
            pltpu.VMEM((LB, PAIR), jnp.float32),
            pltpu.VMEM((FT * TILE_E,), jnp.float32),
            pltpu.VMEM((FT * TILE_E,), jnp.float32),
            pltpu.SemaphoreType.DMA,
            pltpu.SemaphoreType.DMA,
            pltpu.SemaphoreType.DMA,
            pltpu.SemaphoreType.DMA,
        ],
    )
    def gather_kernel(xt_hbm, table_hbm, out_hbm, idx_all, idx_half,
                      src0, src1, dst0, dst1, gsem0, gsem1, osem0, osem1):
        wid = lax.axis_index("s") * NC + lax.axis_index("c")
        srcs = (src0, src1)
        dsts = (dst0, dst1)
        gsems = (gsem0, gsem1)
        osems = (osem0, osem1)
        lane = jax.lax.iota(jnp.int32, LANES)
        perms = [(lane + k) % LANES for k in range(LANES)]
        dst_base = [perms[k] * LB + lane for k in range(LANES)]

        def start_gather(n, b):
            pltpu.async_copy(
                table_hbm.at[idx_half.at[n // 8, n % 8]], srcs[b], gsems[b]
            )

        def wait_gather(b):
            pltpu.make_async_copy(
                table_hbm.at[pl.ds(0, LB)], srcs[b], gsems[b]
            ).wait()

        def wait_out(b):
            for ft in range(FT):
                pltpu.make_async_copy(
                    dsts[b].at[pl.ds(ft * TILE_E, TILE_E)],
                    out_hbm.at[0, ft, 0],
                    osems[b],
                ).wait()

        def transpose_scale(n, b):
            src, dst = srcs[b], dsts[b]
            st, si = n // 8, n % 8
            n_f = D_MODEL // LANES

            @plsc.parallel_loop(0, (LB // LANES) * n_f, unroll=2)
            def _(t):
                b0 = (t // n_f) * LANES
                f0 = (t % n_f) * LANES
                row_idx = lane + b0
                par64 = (idx_all[st, si, pl.ds(b0, LANES)] & 1) * D_MODEL
                colb = par64 + f0
                dbase = f0 * LB + b0
                for k in range(LANES):
                    vec = plsc.load_gather(src, [row_idx, colb + perms[k]])
                    plsc.store_scatter(dst, [dst_base[k] + dbase], vec * SCALE)

        def start_out(n, b):
            for ft in range(FT):
                pltpu.async_copy(
                    dsts[b].at[pl.ds(ft * TILE_E, TILE_E)],
                    out_hbm.at[n, ft, wid],
                    osems[b],
                )

        pltpu.sync_copy(xt_hbm.at[:, wid], idx_all)

        @plsc.parallel_loop(0, seq, unroll=2)
        def _(r):
            st, si = r // 8, r % 8
            for c in range(LB // LANES):
                sl = pl.ds(c * LANES, LANES)
                idx_half[st, si, sl] = idx_all[st, si, sl] >> 1

        start_gather(0, 0)

        def outer(m2, carry):
            for b in range(2):
                n = m2 * 2 + b
                if b == 0:
                    @pl.when(m2 >= 1)
                    def _():
                        wait_out(1 - b)
                    start_gather(n + 1, 1 - b)
                else:
                    @pl.when(m2 <= seq // 2 - 2)
                    def _():
                        wait_out(1 - b)
                        start_gather(n + 1, 1 - b)
                wait_gather(b)
                transpose_scale(n, b)
                start_out(n, b)
            return carry

        lax.fori_loop(0, seq // 2, outer, None)
        wait_out(0)
        wait_out(1)

    return gather_kernel


def kernel(x, table):
    s0, s1 = x.shape
    vocab = table.shape[0]
    x4 = (
        x.astype(jnp.int32)
        .reshape(s0 // LB, LB, s1 // 8, 8)
        .transpose(2, 0, 3, 1)
    )
    t2 = table.reshape(vocab // 2, PAIR)
    out5 = _make_gather(s1, s0, vocab)(x4, t2)
    return (
        out5.reshape(s1, FT, s0 // LB, FT, LB)
        .transpose(2, 4, 0, 1, 3)
        .reshape(s0, s1, D_MODEL)
    )


# speedup vs baseline: 1.4641x; 1.4641x over previous
import functools

import jax
import jax.numpy as jnp
from jax import lax
from jax.experimental import pallas as pl
from jax.experimental.pallas import tpu as pltpu
from jax.experimental.pallas import tpu_sc as plsc

D_MODEL = 64
SCALE = 8.0
LANES = 16
LB = 128
PAIR = 2 * D_MODEL
FT = 8
TILE_E = FT * 128


@functools.cache
def _make_detile(vocab: int):
    info = plsc.get_sparse_core_info()
    NC, NS = info.num_cores, info.num_subcores
    NW = NC * NS
    n_full = vocab // LB
    tail = vocab - n_full * LB
    per_w = (n_full + NW - 1) // NW
    per_w += per_w % 2

    mesh = plsc.VectorSubcoreMesh(core_axis_name="c", subcore_axis_name="s")

    @functools.partial(
        pl.kernel,
        mesh=mesh,
        compiler_params=pltpu.CompilerParams(needs_layout_passes=False),
        out_type=jax.ShapeDtypeStruct((vocab // 2, PAIR), jnp.float32),
        scratch_types=[
            pltpu.VMEM((D_MODEL, LB), jnp.float32),
            pltpu.VMEM((D_MODEL, LB), jnp.float32),
            pltpu.VMEM((D_MODEL, LB), jnp.float32),
            pltpu.VMEM((D_MODEL, LB), jnp.float32),
            pltpu.SemaphoreType.DMA,
            pltpu.SemaphoreType.DMA,
            pltpu.SemaphoreType.DMA,
            pltpu.SemaphoreType.DMA,
        ],
    )
    def detile_kernel(tt_hbm, ttail_hbm, t2_hbm, in0, in1, o0, o1,
                      isem0, isem1, osem0, osem1):
        wid = lax.axis_index("s") * NC + lax.axis_index("c")
        ins = (in0, in1)
        outs = (o0, o1)
        isems = (isem0, isem1)
        osems = (osem0, osem1)
        lane = jax.lax.iota(jnp.int32, LANES)
        perms = [(lane + k) % LANES for k in range(LANES)]
        rowb = lane // 2
        colb = [(lane % 2) * D_MODEL + perms[k] for k in range(LANES)]

        def blk_of(j):
            return jnp.minimum(wid + NW * j, n_full - 1)

        def start_in(j, b):
            v0 = blk_of(j) * LB
            pltpu.async_copy(tt_hbm.at[:, pl.ds(v0, LB)], ins[b], isems[b])

        def wait_in(b):
            pltpu.make_async_copy(
                tt_hbm.at[:, pl.ds(0, LB)], ins[b], isems[b]
            ).wait()

        def wait_out(b):
            pltpu.make_async_copy(
                outs[b], t2_hbm.at[pl.ds(0, D_MODEL)], osems[b]
            ).wait()

        def transpose(b, nv=LB // LANES):
            src, dst = ins[b], outs[b]
            n_f = D_MODEL // LANES

            @plsc.parallel_loop(0, nv * n_f, unroll=2)
            def _(t):
                vi0 = (t // n_f) * LANES
                f0 = (t % n_f) * LANES
                col = vi0 + lane
                row2 = rowb + (vi0 // 2)
                for k in range(LANES):
                    vec = plsc.load_gather(src, [perms[k] + f0, col])
                    plsc.store_scatter(dst, [row2, colb[k] + f0], vec)

        def start_out(j, b):
            p0 = blk_of(j) * (LB // 2)
            pltpu.async_copy(
                outs[b], t2_hbm.at[pl.ds(p0, D_MODEL)], osems[b]
            )

        start_in(0, 0)

        def outer(m2, carry):
            for b in range(2):
                j = m2 * 2 + b
                if b == 0:
                    @pl.when(m2 >= 1)
                    def _():
                        wait_out(1 - b)
                    start_in(j + 1, 1 - b)
                else:
                    @pl.when(m2 <= per_w // 2 - 2)
                    def _():
                        wait_out(1 - b)
                        start_in(j + 1, 1 - b)
                wait_in(b)
                transpose(b)
                start_out(j, b)
            return carry

        lax.fori_loop(0, per_w // 2, outer, None)
        wait_out(0)
        wait_out(1)

        if tail:
            @pl.when(wid == 0)
            def _():
                pltpu.sync_copy(ttail_hbm, ins[0])
                transpose(0, nv=tail // LANES)
                pltpu.sync_copy(
                    outs[0].at[pl.ds(0, tail // 2)],
                    t2_hbm.at[pl.ds((n_full * LB) // 2, tail // 2)],
                )

    return detile_kernel


@functools.cache
def _make_gather(seq: int, batch: int, vocab: int):
    info = plsc.get_sparse_core_info()
    NC, NS = info.num_cores, info.num_subcores
    NW = NC * NS
    n_bt = batch // LB
    assert n_bt == NW and seq % 2 == 0 and vocab % 2 == 0, (seq, batch, vocab)

    mesh = plsc.VectorSubcoreMesh(core_axis_name="c", subcore_axis_name="s")

    @functools.partial(
        pl.kernel,
        mesh=mesh,
        compiler_params=pltpu.CompilerParams(
            use_tc_tiling_on_sc=False, needs_layout_passes=False
        ),
        out_type=jax.ShapeDtypeStruct((seq, FT, n_bt, TILE_E), jnp.float32),
        scratch_types=[
            pltpu.VMEM((seq // 8, 8, LB), jnp.int32),
            pltpu.VMEM((seq // 8, 8, LB), jnp.int32),
            pltpu.VMEM((LB, PAIR), jnp.float32),
            pltpu.VMEM((LB, PAIR), jnp.float32),
            pltpu.VMEM((FT * TILE_E,), jnp.float32),
            pltpu.VMEM((FT * TILE_E,), jnp.float32),
            pltpu.SemaphoreType.DMA,
            pltpu.SemaphoreType.DMA,
            pltpu.SemaphoreType.DMA,
            pltpu.SemaphoreType.DMA,
        ],
    )
    def gather_kernel(xt_hbm, table_hbm, out_hbm, idx_all, idx_half,
                      src0, src1, dst0, dst1, gsem0, gsem1, osem0, osem1):
        wid = lax.axis_index("s") * NC + lax.axis_index("c")
        srcs = (src0, src1)
        dsts = (dst0, dst1)
        gsems = (gsem0, gsem1)
        osems = (osem0, osem1)
        lane = jax.lax.iota(jnp.int32, LANES)
        perms = [(lane + k) % LANES for k in range(LANES)]
        dst_base = [perms[k] * LB + lane for k in range(LANES)]

        def start_gather(n, b):
            pltpu.async_copy(
                table_hbm.at[idx_half.at[n // 8, n % 8]], srcs[b], gsems[b]
            )

        def wait_gather(b):
            pltpu.make_async_copy(
                table_hbm.at[pl.ds(0, LB)], srcs[b], gsems[b]
            ).wait()

        def wait_out(b):
            for ft in range(FT):
                pltpu.make_async_copy(
                    dsts[b].at[pl.ds(ft * TILE_E, TILE_E)],
                    out_hbm.at[0, ft, 0],
                    osems[b],
                ).wait()

        def transpose_scale(n, b):
            src, dst = srcs[b], dsts[b]
            st, si = n // 8, n % 8
            n_f = D_MODEL // LANES

            @plsc.parallel_loop(0, (LB // LANES) * n_f, unroll=2)
            def _(t):
                b0 = (t // n_f) * LANES
                f0 = (t % n_f) * LANES
                row_idx = lane + b0
                par64 = (idx_all[st, si, pl.ds(b0, LANES)] & 1) * D_MODEL
                colb = par64 + f0
                dbase = f0 * LB + b0
                for k in range(LANES):
                    vec = plsc.load_gather(src, [row_idx, colb + perms[k]])
                    plsc.store_scatter(dst, [dst_base[k] + dbase], vec * SCALE)

        def start_out(n, b):
            for ft in range(FT):
                pltpu.async_copy(
                    dsts[b].at[pl.ds(ft * TILE_E, TILE_E)],
                    out_hbm.at[n, ft, wid],
                    osems[b],
                )

        pltpu.sync_copy(xt_hbm.at[:, wid], idx_all)

        @plsc.parallel_loop(0, seq, unroll=2)
        def _(r):
            st, si = r // 8, r % 8
            for c in range(LB // LANES):
                sl = pl.ds(c * LANES, LANES)
                idx_half[st, si, sl] = idx_all[st, si, sl] >> 1

        start_gather(0, 0)

        def outer(m2, carry):
            for b in range(2):
                n = m2 * 2 + b
                if b == 0:
                    @pl.when(m2 >= 1)
                    def _():
                        wait_out(1 - b)
                    start_gather(n + 1, 1 - b)
                else:
                    @pl.when(m2 <= seq // 2 - 2)
                    def _():
                        wait_out(1 - b)
                        start_gather(n + 1, 1 - b)
                wait_gather(b)
                transpose_scale(n, b)
                start_out(n, b)
            return carry

        lax.fori_loop(0, seq // 2, outer, None)
        wait_out(0)
        wait_out(1)

    return gather_kernel


def kernel(x, table):
    s0, s1 = x.shape
    vocab = table.shape[0]
    x4 = (
        x.astype(jnp.int32)
        .reshape(s0 // LB, LB, s1 // 8, 8)
        .transpose(2, 0, 3, 1)
    )
    n_full = table.shape[0] // LB
    ttail = jnp.transpose(
        jnp.pad(table[n_full * LB :], ((0, LB - (vocab - n_full * LB)), (0, 0)))
    )
    t2 = _make_detile(vocab)(jnp.transpose(table), ttail)
    out5 = _make_gather(s1, s0, vocab)(x4, t2)
    return (
        out5.reshape(s1, FT, s0 // LB, FT, LB)
        .transpose(2, 4, 0, 1, 3)
        .reshape(s0, s1, D_MODEL)
    )


# speedup vs baseline: 1.6269x; 1.1112x over previous
import functools

import jax
import jax.numpy as jnp
from jax import lax
from jax.experimental import pallas as pl
from jax.experimental.pallas import tpu as pltpu
from jax.experimental.pallas import tpu_sc as plsc

D_MODEL = 64
SCALE = 8.0
LANES = 16
LB = 128
PAIR = 2 * D_MODEL
FT = 8
TILE_E = FT * 128


@functools.cache
def _make_detile(vocab: int):
    info = plsc.get_sparse_core_info()
    NC, NS = info.num_cores, info.num_subcores
    NW = NC * NS
    n_full = vocab // LB
    tail = vocab - n_full * LB
    per_w = (n_full + NW - 1) // NW
    per_w += per_w % 2

    mesh = plsc.VectorSubcoreMesh(core_axis_name="c", subcore_axis_name="s")

    @functools.partial(
        pl.kernel,
        mesh=mesh,
        compiler_params=pltpu.CompilerParams(needs_layout_passes=False),
        out_type=jax.ShapeDtypeStruct((vocab // 2, PAIR), jnp.float32),
        scratch_types=[
            pltpu.VMEM((D_MODEL, LB), jnp.float32),
            pltpu.VMEM((D_MODEL, LB), jnp.float32),
            pltpu.VMEM((D_MODEL, LB), jnp.float32),
            pltpu.VMEM((D_MODEL, LB), jnp.float32),
            pltpu.SemaphoreType.DMA,
            pltpu.SemaphoreType.DMA,
            pltpu.SemaphoreType.DMA,
            pltpu.SemaphoreType.DMA,
        ],
    )
    def detile_kernel(tt_hbm, ttail_hbm, t2_hbm, in0, in1, o0, o1,
                      isem0, isem1, osem0, osem1):
        wid = lax.axis_index("s") * NC + lax.axis_index("c")
        ins = (in0, in1)
        outs = (o0, o1)
        isems = (isem0, isem1)
        osems = (osem0, osem1)
        lane = jax.lax.iota(jnp.int32, LANES)
        perms = [(lane + k) % LANES for k in range(LANES)]
        zero = lane * 0
        srcb = [perms[k] * LB + lane for k in range(LANES)]
        dstb = [lane * D_MODEL + perms[k] for k in range(LANES)]

        def blk_of(j):
            return jnp.minimum(wid + NW * j, n_full - 1)

        def start_in(j, b):
            v0 = blk_of(j) * LB
            pltpu.async_copy(tt_hbm.at[:, pl.ds(v0, LB)], ins[b], isems[b])

        def wait_in(b):
            pltpu.make_async_copy(
                tt_hbm.at[:, pl.ds(0, LB)], ins[b], isems[b]
            ).wait()

        def wait_out(b):
            pltpu.make_async_copy(
                outs[b], t2_hbm.at[pl.ds(0, D_MODEL)], osems[b]
            ).wait()

        def transpose(b, nv=LB // LANES):
            src, dst = ins[b], outs[b]
            n_f = D_MODEL // LANES

            @plsc.parallel_loop(0, nv * n_f, unroll=2)
            def _(t):
                vi0 = (t // n_f) * LANES
                f0 = (t % n_f) * LANES
                soff = f0 * LB + vi0
                doff = vi0 * D_MODEL + f0
                for k in range(LANES):
                    vec = plsc.load_gather(src, [zero, srcb[k] + soff])
                    plsc.store_scatter(dst, [zero, dstb[k] + doff], vec)

        def start_out(j, b):
            p0 = blk_of(j) * (LB // 2)
            pltpu.async_copy(
                outs[b], t2_hbm.at[pl.ds(p0, D_MODEL)], osems[b]
            )

        start_in(0, 0)

        def outer(m2, carry):
            for b in range(2):
                j = m2 * 2 + b
                if b == 0:
                    @pl.when(m2 >= 1)
                    def _():
                        wait_out(1 - b)
                    start_in(j + 1, 1 - b)
                else:
                    @pl.when(m2 <= per_w // 2 - 2)
                    def _():
                        wait_out(1 - b)
                        start_in(j + 1, 1 - b)
                wait_in(b)
                transpose(b)
                start_out(j, b)
            return carry

        lax.fori_loop(0, per_w // 2, outer, None)
        wait_out(0)
        wait_out(1)

        if tail:
            @pl.when(wid == 0)
            def _():
                pltpu.sync_copy(ttail_hbm, ins[0])
                transpose(0, nv=tail // LANES)
                pltpu.sync_copy(
                    outs[0].at[pl.ds(0, tail // 2)],
                    t2_hbm.at[pl.ds((n_full * LB) // 2, tail // 2)],
                )

    return detile_kernel


@functools.cache
def _make_gather(seq: int, batch: int, vocab: int):
    info = plsc.get_sparse_core_info()
    NC, NS = info.num_cores, info.num_subcores
    NW = NC * NS
    n_bt = batch // LB
    assert n_bt == NW and seq % 2 == 0 and vocab % 2 == 0, (seq, batch, vocab)

    mesh = plsc.VectorSubcoreMesh(core_axis_name="c", subcore_axis_name="s")

    @functools.partial(
        pl.kernel,
        mesh=mesh,
        compiler_params=pltpu.CompilerParams(
            use_tc_tiling_on_sc=False, needs_layout_passes=False
        ),
        out_type=jax.ShapeDtypeStruct((seq, FT, n_bt, TILE_E), jnp.float32),
        scratch_types=[
            pltpu.VMEM((seq // 8, 8, LB), jnp.int32),
            pltpu.VMEM((seq // 8, 8, LB), jnp.int32),
            pltpu.VMEM((LB, PAIR), jnp.float32),
            pltpu.VMEM((LB, PAIR), jnp.float32),
            pltpu.VMEM((FT * TILE_E,), jnp.float32),
            pltpu.VMEM((FT * TILE_E,), jnp.float32),
            pltpu.SemaphoreType.DMA,
            pltpu.SemaphoreType.DMA,
            pltpu.SemaphoreType.DMA,
            pltpu.SemaphoreType.DMA,
        ],
    )
    def gather_kernel(xt_hbm, table_hbm, out_hbm, idx_all, idx_half,
                      src0, src1, dst0, dst1, gsem0, gsem1, osem0, osem1):
        wid = lax.axis_index("s") * NC + lax.axis_index("c")
        srcs = (src0, src1)
        dsts = (dst0, dst1)
        gsems = (gsem0, gsem1)
        osems = (osem0, osem1)
        lane = jax.lax.iota(jnp.int32, LANES)
        perms = [(lane + k) % LANES for k in range(LANES)]
        zero = lane * 0
        src_base = [lane * PAIR + perms[k] for k in range(LANES)]
        dst_base = [perms[k] * LB + lane for k in range(LANES)]

        def start_gather(n, b):
            pltpu.async_copy(
                table_hbm.at[idx_half.at[n // 8, n % 8]], srcs[b], gsems[b]
            )

        def wait_gather(b):
            pltpu.make_async_copy(
                table_hbm.at[pl.ds(0, LB)], srcs[b], gsems[b]
            ).wait()

        def wait_out(b):
            for ft in range(FT):
                pltpu.make_async_copy(
                    dsts[b].at[pl.ds(ft * TILE_E, TILE_E)],
                    out_hbm.at[0, ft, 0],
                    osems[b],
                ).wait()

        def transpose_scale(n, b):
            src, dst = srcs[b], dsts[b]
            st, si = n // 8, n % 8
            n_f = D_MODEL // LANES

            @plsc.parallel_loop(0, (LB // LANES) * n_f, unroll=2)
            def _(t):
                b0 = (t // n_f) * LANES
                f0 = (t % n_f) * LANES
                par64 = (idx_all[st, si, pl.ds(b0, LANES)] & 1) * D_MODEL
                soff = par64 + (b0 * PAIR + f0)
                dbase = f0 * LB + b0
                for k in range(LANES):
                    vec = plsc.load_gather(src, [zero, src_base[k] + soff])
                    plsc.store_scatter(dst, [dst_base[k] + dbase], vec * SCALE)

        def start_out(n, b):
            for ft in range(FT):
                pltpu.async_copy(
                    dsts[b].at[pl.ds(ft * TILE_E, TILE_E)],
                    out_hbm.at[n, ft, wid],
                    osems[b],
                )

        pltpu.sync_copy(xt_hbm.at[:, wid], idx_all)

        @plsc.parallel_loop(0, seq, unroll=2)
        def _(r):
            st, si = r // 8, r % 8
            for c in range(LB // LANES):
                sl = pl.ds(c * LANES, LANES)
                idx_half[st, si, sl] = idx_all[st, si, sl] >> 1

        start_gather(0, 0)

        def outer(m2, carry):
            for b in range(2):
                n = m2 * 2 + b
                if b == 0:
                    @pl.when(m2 >= 1)
                    def _():
                        wait_out(1 - b)
                    start_gather(n + 1, 1 - b)
                else:
                    @pl.when(m2 <= seq // 2 - 2)
                    def _():
                        wait_out(1 - b)
                        start_gather(n + 1, 1 - b)
                wait_gather(b)
                transpose_scale(n, b)
                start_out(n, b)
            return carry

        lax.fori_loop(0, seq // 2, outer, None)
        wait_out(0)
        wait_out(1)

    return gather_kernel


def kernel(x, table):
    s0, s1 = x.shape
    vocab = table.shape[0]
    x4 = (
        x.astype(jnp.int32)
        .reshape(s0 // LB, LB, s1 // 8, 8)
        .transpose(2, 0, 3, 1)
    )
    n_full = table.shape[0] // LB
    ttail = jnp.transpose(
        jnp.pad(table[n_full * LB :], ((0, LB - (vocab - n_full * LB)), (0, 0)))
    )
    t2 = _make_detile(vocab)(jnp.transpose(table), ttail)
    out5 = _make_gather(s1, s0, vocab)(x4, t2)
    return (
        out5.reshape(s1, FT, s0 // LB, FT, LB)
        .transpose(2, 4, 0, 1, 3)
        .reshape(s0, s1, D_MODEL)
    )


# speedup vs baseline: 1.6552x; 1.0174x over previous
import functools

import jax
import jax.numpy as jnp
from jax import lax
from jax.experimental import pallas as pl
from jax.experimental.pallas import tpu as pltpu
from jax.experimental.pallas import tpu_sc as plsc

D_MODEL = 64
SCALE = 8.0
LANES = 16
LB = 128
PAIR = 2 * D_MODEL
FT = 8
TILE_E = FT * 128


@functools.cache
def _make_detile(vocab: int):
    info = plsc.get_sparse_core_info()
    NC, NS = info.num_cores, info.num_subcores
    NW = NC * NS
    n_full = vocab // LB
    tail = vocab - n_full * LB
    per_w = (n_full + NW - 1) // NW
    per_w += per_w % 2

    mesh = plsc.VectorSubcoreMesh(core_axis_name="c", subcore_axis_name="s")

    @functools.partial(
        pl.kernel,
        mesh=mesh,
        compiler_params=pltpu.CompilerParams(needs_layout_passes=False),
        out_type=jax.ShapeDtypeStruct((vocab // 2, PAIR), jnp.float32),
        scratch_types=[
            pltpu.VMEM((D_MODEL, LB), jnp.float32),
            pltpu.VMEM((D_MODEL, LB), jnp.float32),
            pltpu.VMEM((D_MODEL, LB), jnp.float32),
            pltpu.VMEM((D_MODEL, LB), jnp.float32),
            pltpu.SemaphoreType.DMA,
            pltpu.SemaphoreType.DMA,
            pltpu.SemaphoreType.DMA,
            pltpu.SemaphoreType.DMA,
        ],
    )
    def detile_kernel(tt_hbm, ttail_hbm, t2_hbm, in0, in1, o0, o1,
                      isem0, isem1, osem0, osem1):
        wid = lax.axis_index("s") * NC + lax.axis_index("c")
        ins = (in0, in1)
        outs = (o0, o1)
        isems = (isem0, isem1)
        osems = (osem0, osem1)
        lane = jax.lax.iota(jnp.int32, LANES)
        perms = [(lane + k) % LANES for k in range(LANES)]
        zero = lane * 0
        srcb = [perms[k] * LB + lane for k in range(LANES)]
        dstb = [lane * D_MODEL + perms[k] for k in range(LANES)]

        def blk_of(j):
            return jnp.minimum(wid + NW * j, n_full - 1)

        def start_in(j, b):
            v0 = blk_of(j) * LB
            pltpu.async_copy(tt_hbm.at[:, pl.ds(v0, LB)], ins[b], isems[b])

        def wait_in(b):
            pltpu.make_async_copy(
                tt_hbm.at[:, pl.ds(0, LB)], ins[b], isems[b]
            ).wait()

        def wait_out(b):
            pltpu.make_async_copy(
                outs[b], t2_hbm.at[pl.ds(0, D_MODEL)], osems[b]
            ).wait()

        def transpose(b, nv=LB // LANES):
            src, dst = ins[b], outs[b]
            n_f = D_MODEL // LANES

            @plsc.parallel_loop(0, nv * n_f, unroll=4)
            def _(t):
                vi0 = (t // n_f) * LANES
                f0 = (t % n_f) * LANES
                soff = f0 * LB + vi0
                doff = vi0 * D_MODEL + f0
                for k in range(LANES):
                    vec = plsc.load_gather(src, [zero, srcb[k] + soff])
                    plsc.store_scatter(dst, [zero, dstb[k] + doff], vec)

        def start_out(j, b):
            p0 = blk_of(j) * (LB // 2)
            pltpu.async_copy(
                outs[b], t2_hbm.at[pl.ds(p0, D_MODEL)], osems[b]
            )

        start_in(0, 0)

        def outer(m2, carry):
            for b in range(2):
                j = m2 * 2 + b
                if b == 0:
                    @pl.when(m2 >= 1)
                    def _():
                        wait_out(1 - b)
                    start_in(j + 1, 1 - b)
                else:
                    @pl.when(m2 <= per_w // 2 - 2)
                    def _():
                        wait_out(1 - b)
                        start_in(j + 1, 1 - b)
                wait_in(b)
                transpose(b)
                start_out(j, b)
            return carry

        lax.fori_loop(0, per_w // 2, outer, None)
        wait_out(0)
        wait_out(1)

        if tail:
            @pl.when(wid == 0)
            def _():
                pltpu.sync_copy(ttail_hbm, ins[0])
                transpose(0, nv=tail // LANES)
                pltpu.sync_copy(
                    outs[0].at[pl.ds(0, tail // 2)],
                    t2_hbm.at[pl.ds((n_full * LB) // 2, tail // 2)],
                )

    return detile_kernel


@functools.cache
def _make_gather(seq: int, batch: int, vocab: int):
    info = plsc.get_sparse_core_info()
    NC, NS = info.num_cores, info.num_subcores
    NW = NC * NS
    n_bt = batch // LB
    assert n_bt == NW and seq % 2 == 0 and vocab % 2 == 0, (seq, batch, vocab)

    mesh = plsc.VectorSubcoreMesh(core_axis_name="c", subcore_axis_name="s")

    @functools.partial(
        pl.kernel,
        mesh=mesh,
        compiler_params=pltpu.CompilerParams(
            use_tc_tiling_on_sc=False, needs_layout_passes=False
        ),
        out_type=jax.ShapeDtypeStruct((seq, FT, n_bt, TILE_E), jnp.float32),
        scratch_types=[
            pltpu.VMEM((seq // 8, 8, LB), jnp.int32),
            pltpu.VMEM((seq // 8, 8, LB), jnp.int32),
            pltpu.VMEM((LB, PAIR), jnp.float32),
            pltpu.VMEM((LB, PAIR), jnp.float32),
            pltpu.VMEM((FT * TILE_E,), jnp.float32),
            pltpu.VMEM((FT * TILE_E,), jnp.float32),
            pltpu.SemaphoreType.DMA,
            pltpu.SemaphoreType.DMA,
            pltpu.SemaphoreType.DMA,
            pltpu.SemaphoreType.DMA,
        ],
    )
    def gather_kernel(xt_hbm, table_hbm, out_hbm, idx_all, idx_half,
                      src0, src1, dst0, dst1, gsem0, gsem1, osem0, osem1):
        wid = lax.axis_index("s") * NC + lax.axis_index("c")
        srcs = (src0, src1)
        dsts = (dst0, dst1)
        gsems = (gsem0, gsem1)
        osems = (osem0, osem1)
        lane = jax.lax.iota(jnp.int32, LANES)
        perms = [(lane + k) % LANES for k in range(LANES)]
        zero = lane * 0
        src_base = [lane * PAIR + perms[k] for k in range(LANES)]
        dst_base = [perms[k] * LB + lane for k in range(LANES)]

        def start_gather(n, b):
            pltpu.async_copy(
                table_hbm.at[idx_half.at[n // 8, n % 8]], srcs[b], gsems[b]
            )

        def wait_gather(b):
            pltpu.make_async_copy(
                table_hbm.at[pl.ds(0, LB)], srcs[b], gsems[b]
            ).wait()

        def wait_out(b):
            for ft in range(FT):
                pltpu.make_async_copy(
                    dsts[b].at[pl.ds(ft * TILE_E, TILE_E)],
                    out_hbm.at[0, ft, 0],
                    osems[b],
                ).wait()

        def transpose_scale(n, b):
            src, dst = srcs[b], dsts[b]
            st, si = n // 8, n % 8
            n_f = D_MODEL // LANES

            @plsc.parallel_loop(0, (LB // LANES) * n_f, unroll=4)
            def _(t):
                b0 = (t // n_f) * LANES
                f0 = (t % n_f) * LANES
                par64 = (idx_all[st, si, pl.ds(b0, LANES)] & 1) * D_MODEL
                soff = par64 + (b0 * PAIR + f0)
                dbase = f0 * LB + b0
                for k in range(LANES):
                    vec = plsc.load_gather(src, [zero, src_base[k] + soff])
                    plsc.store_scatter(dst, [dst_base[k] + dbase], vec * SCALE)

        def start_out(n, b):
            for ft in range(FT):
                pltpu.async_copy(
                    dsts[b].at[pl.ds(ft * TILE_E, TILE_E)],
                    out_hbm.at[n, ft, wid],
                    osems[b],
                )

        pltpu.sync_copy(xt_hbm.at[:, wid], idx_all)

        @plsc.parallel_loop(0, seq, unroll=2)
        def _(r):
            st, si = r // 8, r % 8
            for c in range(LB // LANES):
                sl = pl.ds(c * LANES, LANES)
                idx_half[st, si, sl] = idx_all[st, si, sl] >> 1

        start_gather(0, 0)

        def outer(m2, carry):
            for b in range(2):
                n = m2 * 2 + b
                if b == 0:
                    @pl.when(m2 >= 1)
                    def _():
                        wait_out(1 - b)
                    start_gather(n + 1, 1 - b)
                else:
                    @pl.when(m2 <= seq // 2 - 2)
                    def _():
                        wait_out(1 - b)
                        start_gather(n + 1, 1 - b)
                wait_gather(b)
                transpose_scale(n, b)
                start_out(n, b)
            return carry

        lax.fori_loop(0, seq // 2, outer, None)
        wait_out(0)
        wait_out(1)

    return gather_kernel


def kernel(x, table):
    s0, s1 = x.shape
    vocab = table.shape[0]
    x4 = (
        x.astype(jnp.int32)
        .reshape(s0 // LB, LB, s1 // 8, 8)
        .transpose(2, 0, 3, 1)
    )
    n_full = table.shape[0] // LB
    ttail = jnp.transpose(
        jnp.pad(table[n_full * LB :], ((0, LB - (vocab - n_full * LB)), (0, 0)))
    )
    t2 = _make_detile(vocab)(jnp.transpose(table), ttail)
    out5 = _make_gather(s1, s0, vocab)(x4, t2)
    return (
        out5.reshape(s1, FT, s0 // LB, FT, LB)
        .transpose(2, 4, 0, 1, 3)
        .reshape(s0, s1, D_MODEL)
    )
